# baseline (speedup 1.0000x reference)
import functools

import jax
import jax.numpy as jnp
from jax import lax
from jax.experimental import pallas as pl
from jax.experimental.pallas import tpu as pltpu
from jax.experimental.pallas import tpu_sc as plsc

N_NODES = 10000
N_EDGES = 80000
D_IN = 64
D_EDGE = 16
D_CONV = 32
D_LSTM = 32
D_OUT = 16

NW = 32
SUB = 128
NSUB = 20
CHUNK = SUB * NSUB
EP = NW * CHUNK
NA = 10240
STRIPE = NA // 16

EPH = EP // 2
NSUBH = NSUB // 2
CHUNKH = SUB * NSUBH


@functools.cache
def _sc_kernels(half):
    mesh = plsc.VectorSubcoreMesh(core_axis_name="c", subcore_axis_name="s",
                                  num_cores=2, num_subcores=16)
    HBASE = half * EPH

    NB = 2
    XSTRIPE = NA // 16
    @functools.partial(
        pl.kernel,
        mesh=mesh,
        out_type=jax.ShapeDtypeStruct((EPH, 128), jnp.float32),
        scratch_types=(
            [pltpu.VMEM((CHUNKH,), jnp.int32)]
            + [pltpu.VMEM((SUB, 128), jnp.float32) for _ in range(NB)]
            + [pltpu.SemaphoreType.DMA for _ in range(2 * NB)]
            + [pltpu.VMEM_SHARED((NA, 128), jnp.float32)]
        ),
    )
    def gather_rows(x_hbm, src_hbm, out_hbm, *scratch):
        idx_v = scratch[0]
        bufs = scratch[1:1 + NB]
        gsems = scratch[1 + NB:1 + 2 * NB]
        osems = scratch[1 + 2 * NB:1 + 3 * NB]
        xs = scratch[1 + 3 * NB]
        c = lax.axis_index("c")
        s = lax.axis_index("s")
        wid = s * 2 + c
        base = wid * CHUNKH
        pltpu.sync_copy(x_hbm.at[pl.ds(s * XSTRIPE, XSTRIPE)],
                        xs.at[pl.ds(s * XSTRIPE, XSTRIPE)])
        pltpu.sync_copy(src_hbm.at[pl.ds(HBASE + base, CHUNKH)], idx_v)
        plsc.subcore_barrier()
        gc = [None] * NB
        oc = [None] * NB
        for j in range(NB):
            gc[j] = pltpu.async_copy(
                xs.at[idx_v.at[pl.ds(j * SUB, SUB)]], bufs[j], gsems[j])
        for j in range(NSUBH):
            sl = j % NB
            gc[sl].wait()
            oc[sl] = pltpu.async_copy(
                bufs[sl], out_hbm.at[pl.ds(base + j * SUB, SUB)], osems[sl])
            nj = j + NB
            if nj < NSUBH:
                oc[sl].wait()
                gc[sl] = pltpu.async_copy(
                    xs.at[idx_v.at[pl.ds(nj * SUB, SUB)]], bufs[sl],
                    gsems[sl])
        for j in range(NSUBH - NB, NSUBH):
            oc[j % NB].wait()

    @functools.partial(
        pl.kernel,
        mesh=mesh,
        out_type=jax.ShapeDtypeStruct((2, NA, 128), jnp.float32),
        scratch_types=[
            pltpu.VMEM((SUB,), jnp.int32),
            pltpu.VMEM((SUB,), jnp.int32),
            pltpu.VMEM((SUB, 128), jnp.float32),
            pltpu.VMEM((SUB, 128), jnp.float32),
            pltpu.SemaphoreType.DMA,
            pltpu.SemaphoreType.DMA,
            pltpu.SemaphoreType.DMA,
            pltpu.SemaphoreType.DMA,
            pltpu.VMEM_SHARED((NA, 128), jnp.float32),
        ],
    )
    def scatter_add(dst_hbm, msg_hbm, zeros_hbm, out_hbm, i0, i1, m0, m1,
                    si0, si1, sm0, sm1, shared):
        ibufs = (i0, i1)
        mbufs = (m0, m1)
        isems = (si0, si1)
        msems = (sm0, sm1)
        c = lax.axis_index("c")
        s = lax.axis_index("s")
        pltpu.sync_copy(zeros_hbm, m0)
        for k in range(STRIPE // SUB):
            pltpu.sync_copy(m0, shared.at[pl.ds(s * STRIPE + k * SUB, SUB)])
        plsc.subcore_barrier()
        wid = s * 2 + c
        base = wid * CHUNKH
        ic = [None, None]
        mc = [None, None]
        ic[0] = pltpu.async_copy(dst_hbm.at[pl.ds(HBASE + base, SUB)], i0, si0)
        mc[0] = pltpu.async_copy(msg_hbm.at[pl.ds(base, SUB)], m0, sm0)
        for j in range(NSUBH):
            sl = j % 2
            if j + 1 < NSUBH:
                nsl = (j + 1) % 2
                off = base + (j + 1) * SUB
                ic[nsl] = pltpu.async_copy(
                    dst_hbm.at[pl.ds(HBASE + off, SUB)],
                    ibufs[nsl], isems[nsl])
                mc[nsl] = pltpu.async_copy(msg_hbm.at[pl.ds(off, SUB)],
                                           mbufs[nsl], msems[nsl])
            ic[sl].wait()
            mc[sl].wait()
            pltpu.sync_copy(mbufs[sl], shared.at[ibufs[sl]], add=True)
        plsc.subcore_barrier()
        pltpu.sync_copy(shared.at[pl.ds(s * STRIPE, STRIPE)],
                        out_hbm.at[c, pl.ds(s * STRIPE, STRIPE)])

    return gather_rows, scatter_add


def _msg_body(ea_ref, xj_ref, wf_ref, r_ref, s_ref, br_ref, out_ref):
    xj = xj_ref[:, :D_IN]
    y = jnp.dot(xj, wf_ref[...], preferred_element_type=jnp.float32)
    a = jnp.dot(ea_ref[...], r_ref[...], preferred_element_type=jnp.float32)
    m = jnp.dot(a * y, s_ref[...], preferred_element_type=jnp.float32)
    m = m + jnp.dot(xj, br_ref[...], preferred_element_type=jnp.float32)
    out_ref[...] = jnp.concatenate(
        [m, jnp.zeros((m.shape[0], 128 - D_CONV), jnp.float32)], axis=1)


def _msg_call(half, ea_p, x_j, wflat, rmat, smat, br):
    be = 4096
    grid = EPH // be
    hoff = half * (EPH // be)
    return pl.pallas_call(
        _msg_body,
        grid=(grid,),
        in_specs=[
            pl.BlockSpec((be, D_EDGE), lambda i: (i + hoff, 0)),
            pl.BlockSpec((be, 128), lambda i: (i, 0)),
            pl.BlockSpec((D_IN, D_EDGE * D_CONV), lambda i: (0, 0)),
            pl.BlockSpec((D_EDGE, D_EDGE * D_CONV), lambda i: (0, 0)),
            pl.BlockSpec((D_EDGE * D_CONV, D_CONV), lambda i: (0, 0)),
            pl.BlockSpec((D_IN, D_CONV), lambda i: (0, 0)),
        ],
        out_specs=pl.BlockSpec((be, 128), lambda i: (i, 0)),
        out_shape=jax.ShapeDtypeStruct((EPH, 128), jnp.float32),
    )(ea_p, x_j, wflat, rmat, smat, br)


def _dense_body(x_ref, init_ref, aga_ref, agb_ref, wroot_ref, bconv_ref,
                wih_ref, whh_ref, bg_ref, whs_ref, bhs_ref, wcs_ref, bcs_ref,
                wfin_ref, bfin_ref, out_ref):
    xb = x_ref[...]
    ag = (aga_ref[0][:, :D_CONV] + aga_ref[1][:, :D_CONV]
          + agb_ref[0][:, :D_CONV] + agb_ref[1][:, :D_CONV])
    conv = (ag
            + jnp.dot(xb, wroot_ref[...], preferred_element_type=jnp.float32)
            + bconv_ref[...])
    g = jnp.maximum(conv, 0.0)
    init = init_ref[...]
    h0 = jnp.dot(init, whs_ref[...],
                 preferred_element_type=jnp.float32) + bhs_ref[...]
    c0 = jnp.dot(init, wcs_ref[...],
                 preferred_element_type=jnp.float32) + bcs_ref[...]
    gates = (jnp.dot(g, wih_ref[...], preferred_element_type=jnp.float32)
             + jnp.dot(h0, whh_ref[...], preferred_element_type=jnp.float32)
             + bg_ref[...])
    i_g = jax.nn.sigmoid(gates[:, 0:32])
    f_g = jax.nn.sigmoid(gates[:, 32:64])
    g_g = jnp.tanh(gates[:, 64:96])
    o_g = jax.nn.sigmoid(gates[:, 96:128])
    c1 = f_g * c0 + i_g * g_g
    h1 = o_g * jnp.tanh(c1)
    out_ref[...] = jnp.dot(h1, wfin_ref[...],
                           preferred_element_type=jnp.float32) + bfin_ref[...]


def _dense_call(x, initial, ag_a, ag_b, wroot, bconv, wih, whh, bg, whs, bhs,
                wcs, bcs, wfin, bfin):
    bn = 2000
    grid = N_NODES // bn
    rep = lambda i: (0, 0)
    return pl.pallas_call(
        _dense_body,
        grid=(grid,),
        in_specs=[
            pl.BlockSpec((bn, D_IN), lambda i: (i, 0)),
            pl.BlockSpec((bn, D_OUT), lambda i: (i, 0)),
            pl.BlockSpec((2, bn, 128), lambda i: (0, i, 0)),
            pl.BlockSpec((2, bn, 128), lambda i: (0, i, 0)),
            pl.BlockSpec((D_IN, D_CONV), rep),
            pl.BlockSpec((1, D_CONV), rep),
            pl.BlockSpec((D_CONV, 4 * D_LSTM), rep),
            pl.BlockSpec((D_LSTM, 4 * D_LSTM), rep),
            pl.BlockSpec((1, 4 * D_LSTM), rep),
            pl.BlockSpec((D_OUT, D_LSTM), rep),
            pl.BlockSpec((1, D_LSTM), rep),
            pl.BlockSpec((D_OUT, D_LSTM), rep),
            pl.BlockSpec((1, D_LSTM), rep),
            pl.BlockSpec((D_LSTM, D_OUT), rep),
            pl.BlockSpec((1, D_OUT), rep),
        ],
        out_specs=pl.BlockSpec((bn, D_OUT), lambda i: (i, 0)),
        out_shape=jax.ShapeDtypeStruct((N_NODES, D_OUT), jnp.float32),
    )(x, initial, ag_a, ag_b, wroot, bconv, wih, whh, bg, whs, bhs, wcs, bcs,
      wfin, bfin)


def kernel(x, edge_index, edge_attr, initial, W_cl, b_cl, W_root, b_conv,
           W_ih, W_hh, b_ih, b_hh, W_hs, b_hs, W_cs, b_cs, W_fin, b_fin):
    src = edge_index[0]
    dst = edge_index[1]
    pad = EP - N_EDGES
    src_p = jnp.pad(src, (0, pad))
    dst_p = jnp.pad(dst, (0, pad), constant_values=N_NODES)
    ea_p = jnp.pad(edge_attr, ((0, pad), (0, 0)))

    wflat = W_cl.reshape(D_EDGE, D_IN, D_CONV).transpose(1, 0, 2) \
                .reshape(D_IN, D_EDGE * D_CONV)
    rmat = jnp.repeat(jnp.eye(D_EDGE, dtype=jnp.float32), D_CONV, axis=1)
    smat = jnp.tile(jnp.eye(D_CONV, dtype=jnp.float32), (D_EDGE, 1))
    br = b_cl.reshape(D_IN, D_CONV)
    zeros = jnp.zeros((SUB, 128), jnp.float32)
    x128 = jnp.pad(x, ((0, NA - N_NODES), (0, 128 - D_IN)))

    gather0, scatter0 = _sc_kernels(0)
    gather1, scatter1 = _sc_kernels(1)
    x_j0 = gather0(x128, src_p)
    x_j1 = gather1(x128, src_p)
    msg0 = _msg_call(0, ea_p, x_j0, wflat, rmat, smat, br)
    msg1 = _msg_call(1, ea_p, x_j1, wflat, rmat, smat, br)
    ag_a = scatter0(dst_p, msg0, zeros)
    ag_b = scatter1(dst_p, msg1, zeros)
    return _dense_call(
        x, initial, ag_a, ag_b, W_root, b_conv.reshape(1, D_CONV), W_ih, W_hh,
        (b_ih + b_hh).reshape(1, 4 * D_LSTM), W_hs, b_hs.reshape(1, D_LSTM),
        W_cs, b_cs.reshape(1, D_LSTM), W_fin, b_fin.reshape(1, D_OUT))


# speedup vs baseline: 1.0170x; 1.0170x over previous
import functools

import jax
import jax.numpy as jnp
from jax import lax
from jax.experimental import pallas as pl
from jax.experimental.pallas import tpu as pltpu
from jax.experimental.pallas import tpu_sc as plsc

N_NODES = 10000
N_EDGES = 80000
D_IN = 64
D_EDGE = 16
D_CONV = 32
D_LSTM = 32
D_OUT = 16

NW = 32
SUB = 128
NSUB = 20
CHUNK = SUB * NSUB
EP = NW * CHUNK
NA = 10240
STRIPE = NA // 16

EPH = EP // 2
NSUBH = NSUB // 2
CHUNKH = SUB * NSUBH


def _mesh():
    return plsc.VectorSubcoreMesh(core_axis_name="c", subcore_axis_name="s",
                                  num_cores=2, num_subcores=16)


@functools.cache
def _gather_kernel():
    mesh = _mesh()

    NB = 2
    XSTRIPE = NA // 16
    @functools.partial(
        pl.kernel,
        mesh=mesh,
        out_type=jax.ShapeDtypeStruct((EP, 128), jnp.float32),
        scratch_types=(
            [pltpu.VMEM((CHUNK,), jnp.int32)]
            + [pltpu.VMEM((SUB, 128), jnp.float32) for _ in range(NB)]
            + [pltpu.SemaphoreType.DMA for _ in range(2 * NB)]
            + [pltpu.VMEM_SHARED((NA, 128), jnp.float32)]
        ),
    )
    def gather_rows(x_hbm, src_hbm, out_hbm, *scratch):
        idx_v = scratch[0]
        bufs = scratch[1:1 + NB]
        gsems = scratch[1 + NB:1 + 2 * NB]
        osems = scratch[1 + 2 * NB:1 + 3 * NB]
        xs = scratch[1 + 3 * NB]
        c = lax.axis_index("c")
        s = lax.axis_index("s")
        wid = s * 2 + c
        base = wid * CHUNK
        pltpu.sync_copy(x_hbm.at[pl.ds(s * XSTRIPE, XSTRIPE)],
                        xs.at[pl.ds(s * XSTRIPE, XSTRIPE)])
        pltpu.sync_copy(src_hbm.at[pl.ds(base, CHUNK)], idx_v)
        plsc.subcore_barrier()
        gc = [None] * NB
        oc = [None] * NB
        for j in range(NB):
            gc[j] = pltpu.async_copy(
                xs.at[idx_v.at[pl.ds(j * SUB, SUB)]], bufs[j], gsems[j])
        for j in range(NSUB):
            sl = j % NB
            gc[sl].wait()
            oc[sl] = pltpu.async_copy(
                bufs[sl], out_hbm.at[pl.ds(base + j * SUB, SUB)], osems[sl])
            nj = j + NB
            if nj < NSUB:
                oc[sl].wait()
                gc[sl] = pltpu.async_copy(
                    xs.at[idx_v.at[pl.ds(nj * SUB, SUB)]], bufs[sl],
                    gsems[sl])
        for j in range(NSUB - NB, NSUB):
            oc[j % NB].wait()
    return gather_rows


@functools.cache
def _scatter_kernel(half):
    mesh = _mesh()
    HBASE = half * EPH

    @functools.partial(
        pl.kernel,
        mesh=mesh,
        out_type=jax.ShapeDtypeStruct((2, NA, 128), jnp.float32),
        scratch_types=[
            pltpu.VMEM((SUB,), jnp.int32),
            pltpu.VMEM((SUB,), jnp.int32),
            pltpu.VMEM((SUB, 128), jnp.float32),
            pltpu.VMEM((SUB, 128), jnp.float32),
            pltpu.SemaphoreType.DMA,
            pltpu.SemaphoreType.DMA,
            pltpu.SemaphoreType.DMA,
            pltpu.SemaphoreType.DMA,
            pltpu.VMEM_SHARED((NA, 128), jnp.float32),
        ],
    )
    def scatter_add(dst_hbm, msg_hbm, zeros_hbm, out_hbm, i0, i1, m0, m1,
                    si0, si1, sm0, sm1, shared):
        ibufs = (i0, i1)
        mbufs = (m0, m1)
        isems = (si0, si1)
        msems = (sm0, sm1)
        c = lax.axis_index("c")
        s = lax.axis_index("s")
        pltpu.sync_copy(zeros_hbm, m0)
        for k in range(STRIPE // SUB):
            pltpu.sync_copy(m0, shared.at[pl.ds(s * STRIPE + k * SUB, SUB)])
        plsc.subcore_barrier()
        wid = s * 2 + c
        base = wid * CHUNKH
        ic = [None, None]
        mc = [None, None]
        ic[0] = pltpu.async_copy(dst_hbm.at[pl.ds(HBASE + base, SUB)], i0, si0)
        mc[0] = pltpu.async_copy(msg_hbm.at[pl.ds(base, SUB)], m0, sm0)
        for j in range(NSUBH):
            sl = j % 2
            if j + 1 < NSUBH:
                nsl = (j + 1) % 2
                off = base + (j + 1) * SUB
                ic[nsl] = pltpu.async_copy(
                    dst_hbm.at[pl.ds(HBASE + off, SUB)],
                    ibufs[nsl], isems[nsl])
                mc[nsl] = pltpu.async_copy(msg_hbm.at[pl.ds(off, SUB)],
                                           mbufs[nsl], msems[nsl])
            ic[sl].wait()
            mc[sl].wait()
            pltpu.sync_copy(mbufs[sl], shared.at[ibufs[sl]], add=True)
        plsc.subcore_barrier()
        pltpu.sync_copy(shared.at[pl.ds(s * STRIPE, STRIPE)],
                        out_hbm.at[c, pl.ds(s * STRIPE, STRIPE)])

    return scatter_add


def _msg_body(ea_ref, xj_ref, wf_ref, r_ref, s_ref, br_ref, out_ref):
    xj = xj_ref[:, :D_IN]
    y = jnp.dot(xj, wf_ref[...], preferred_element_type=jnp.float32)
    a = jnp.dot(ea_ref[...], r_ref[...], preferred_element_type=jnp.float32)
    m = jnp.dot(a * y, s_ref[...], preferred_element_type=jnp.float32)
    m = m + jnp.dot(xj, br_ref[...], preferred_element_type=jnp.float32)
    out_ref[...] = jnp.concatenate(
        [m, jnp.zeros((m.shape[0], 128 - D_CONV), jnp.float32)], axis=1)


def _msg_call(half, ea_p, x_j, wflat, rmat, smat, br):
    be = 4096
    grid = EPH // be
    hoff = half * (EPH // be)
    return pl.pallas_call(
        _msg_body,
        grid=(grid,),
        in_specs=[
            pl.BlockSpec((be, D_EDGE), lambda i: (i + hoff, 0)),
            pl.BlockSpec((be, 128), lambda i: (i + hoff, 0)),
            pl.BlockSpec((D_IN, D_EDGE * D_CONV), lambda i: (0, 0)),
            pl.BlockSpec((D_EDGE, D_EDGE * D_CONV), lambda i: (0, 0)),
            pl.BlockSpec((D_EDGE * D_CONV, D_CONV), lambda i: (0, 0)),
            pl.BlockSpec((D_IN, D_CONV), lambda i: (0, 0)),
        ],
        out_specs=pl.BlockSpec((be, 128), lambda i: (i, 0)),
        out_shape=jax.ShapeDtypeStruct((EPH, 128), jnp.float32),
    )(ea_p, x_j, wflat, rmat, smat, br)


def _dense_body(x_ref, init_ref, aga_ref, agb_ref, wroot_ref, bconv_ref,
                wih_ref, whh_ref, bg_ref, whs_ref, bhs_ref, wcs_ref, bcs_ref,
                wfin_ref, bfin_ref, out_ref):
    xb = x_ref[...]
    ag = (aga_ref[0][:, :D_CONV] + aga_ref[1][:, :D_CONV]
          + agb_ref[0][:, :D_CONV] + agb_ref[1][:, :D_CONV])
    conv = (ag
            + jnp.dot(xb, wroot_ref[...], preferred_element_type=jnp.float32)
            + bconv_ref[...])
    g = jnp.maximum(conv, 0.0)
    init = init_ref[...]
    h0 = jnp.dot(init, whs_ref[...],
                 preferred_element_type=jnp.float32) + bhs_ref[...]
    c0 = jnp.dot(init, wcs_ref[...],
                 preferred_element_type=jnp.float32) + bcs_ref[...]
    gates = (jnp.dot(g, wih_ref[...], preferred_element_type=jnp.float32)
             + jnp.dot(h0, whh_ref[...], preferred_element_type=jnp.float32)
             + bg_ref[...])
    i_g = jax.nn.sigmoid(gates[:, 0:32])
    f_g = jax.nn.sigmoid(gates[:, 32:64])
    g_g = jnp.tanh(gates[:, 64:96])
    o_g = jax.nn.sigmoid(gates[:, 96:128])
    c1 = f_g * c0 + i_g * g_g
    h1 = o_g * jnp.tanh(c1)
    out_ref[...] = jnp.dot(h1, wfin_ref[...],
                           preferred_element_type=jnp.float32) + bfin_ref[...]


def _dense_call(x, initial, ag_a, ag_b, wroot, bconv, wih, whh, bg, whs, bhs,
                wcs, bcs, wfin, bfin):
    bn = 2000
    grid = N_NODES // bn
    rep = lambda i: (0, 0)
    return pl.pallas_call(
        _dense_body,
        grid=(grid,),
        in_specs=[
            pl.BlockSpec((bn, D_IN), lambda i: (i, 0)),
            pl.BlockSpec((bn, D_OUT), lambda i: (i, 0)),
            pl.BlockSpec((2, bn, 128), lambda i: (0, i, 0)),
            pl.BlockSpec((2, bn, 128), lambda i: (0, i, 0)),
            pl.BlockSpec((D_IN, D_CONV), rep),
            pl.BlockSpec((1, D_CONV), rep),
            pl.BlockSpec((D_CONV, 4 * D_LSTM), rep),
            pl.BlockSpec((D_LSTM, 4 * D_LSTM), rep),
            pl.BlockSpec((1, 4 * D_LSTM), rep),
            pl.BlockSpec((D_OUT, D_LSTM), rep),
            pl.BlockSpec((1, D_LSTM), rep),
            pl.BlockSpec((D_OUT, D_LSTM), rep),
            pl.BlockSpec((1, D_LSTM), rep),
            pl.BlockSpec((D_LSTM, D_OUT), rep),
            pl.BlockSpec((1, D_OUT), rep),
        ],
        out_specs=pl.BlockSpec((bn, D_OUT), lambda i: (i, 0)),
        out_shape=jax.ShapeDtypeStruct((N_NODES, D_OUT), jnp.float32),
    )(x, initial, ag_a, ag_b, wroot, bconv, wih, whh, bg, whs, bhs, wcs, bcs,
      wfin, bfin)


def kernel(x, edge_index, edge_attr, initial, W_cl, b_cl, W_root, b_conv,
           W_ih, W_hh, b_ih, b_hh, W_hs, b_hs, W_cs, b_cs, W_fin, b_fin):
    src = edge_index[0]
    dst = edge_index[1]
    pad = EP - N_EDGES
    src_p = jnp.pad(src, (0, pad))
    dst_p = jnp.pad(dst, (0, pad), constant_values=N_NODES)
    ea_p = jnp.pad(edge_attr, ((0, pad), (0, 0)))

    wflat = W_cl.reshape(D_EDGE, D_IN, D_CONV).transpose(1, 0, 2) \
                .reshape(D_IN, D_EDGE * D_CONV)
    rmat = jnp.repeat(jnp.eye(D_EDGE, dtype=jnp.float32), D_CONV, axis=1)
    smat = jnp.tile(jnp.eye(D_CONV, dtype=jnp.float32), (D_EDGE, 1))
    br = b_cl.reshape(D_IN, D_CONV)
    zeros = jnp.zeros((SUB, 128), jnp.float32)
    x128 = jnp.pad(x, ((0, NA - N_NODES), (0, 128 - D_IN)))

    gather_rows = _gather_kernel()
    scatter0 = _scatter_kernel(0)
    scatter1 = _scatter_kernel(1)
    x_j = gather_rows(x128, src_p)
    msg0 = _msg_call(0, ea_p, x_j, wflat, rmat, smat, br)
    msg1 = _msg_call(1, ea_p, x_j, wflat, rmat, smat, br)
    ag_a = scatter0(dst_p, msg0, zeros)
    ag_b = scatter1(dst_p, msg1, zeros)
    return _dense_call(
        x, initial, ag_a, ag_b, W_root, b_conv.reshape(1, D_CONV), W_ih, W_hh,
        (b_ih + b_hh).reshape(1, 4 * D_LSTM), W_hs, b_hs.reshape(1, D_LSTM),
        W_cs, b_cs.reshape(1, D_LSTM), W_fin, b_fin.reshape(1, D_OUT))
